# baseline (speedup 1.0000x reference)
import functools

import jax
import jax.numpy as jnp
from jax import lax
from jax.experimental import pallas as pl
from jax.experimental.pallas import tpu as pltpu
from jax.experimental.pallas import tpu_sc as plsc

VOCAB = 150528
HIDDEN = 12288
TOKENS = 8192
TC_TOKENS = 2048
SC_TOKENS = TOKENS - TC_TOKENS

NC, NS = 2, 16
NW = NC * NS
ROWS_PER_W = SC_TOKENS // NW
NG = ROWS_PER_W // 16

_mesh = plsc.VectorSubcoreMesh(
    core_axis_name="c", subcore_axis_name="s", num_cores=NC, num_subcores=NS
)


@functools.partial(
    pl.kernel,
    mesh=_mesh,
    out_type=jax.ShapeDtypeStruct((SC_TOKENS, HIDDEN), jnp.float32),
    scratch_types=[
        pltpu.VMEM((NG, 16), jnp.int32),
        pltpu.VMEM_SHARED((NS, 2, 4, HIDDEN), jnp.float32),
        [pltpu.SemaphoreType.DMA for _ in range(2)],
        [pltpu.SemaphoreType.DMA for _ in range(2)],
    ],
)
def _sc_gather(idx_hbm, table_hbm, out_hbm, idx_v, spbuf, gsem, wsem):
    cid = lax.axis_index("c")
    sid = lax.axis_index("s")
    wid = sid * NC + cid
    base = wid * ROWS_PER_W
    pltpu.sync_copy(idx_hbm.at[wid], idx_v)

    def gather_start(row, b, j):
        pltpu.make_async_copy(
            table_hbm.at[row], spbuf.at[sid, b, j], gsem[b]
        ).start()

    def gather_wait_set(b):
        pltpu.make_async_copy(
            table_hbm.at[pl.ds(0, 4)], spbuf.at[sid, b], gsem[b]
        ).wait()

    def write_desc(r0, b):
        return pltpu.make_async_copy(
            spbuf.at[sid, b], out_hbm.at[pl.ds(base + r0, 4)], wsem[b]
        )

    def chunk_gather(v, lane_cl, cl):
        b = cl % 2
        for j in range(4):
            gather_start(v[4 * lane_cl + j], b, j)

    v0 = idx_v.at[0][...]
    chunk_gather(v0, 0, 0)
    chunk_gather(v0, 1, 1)

    def body(g, carry):
        v = idx_v.at[g][...]
        vn = idx_v.at[g + 1][...]
        for pair in range(2):
            for b in range(2):
                cl = 2 * pair + b
                r0 = 16 * g + 4 * cl
                gather_wait_set(b)
                write_desc(r0, b).start()
            for b in range(2):
                cl = 2 * pair + b
                r0 = 16 * g + 4 * cl
                write_desc(r0, b).wait()
                if cl < 2:
                    chunk_gather(v, cl + 2, cl + 2)
                else:
                    chunk_gather(vn, cl - 2, cl - 2)
        return carry

    lax.fori_loop(0, NG - 1, body, 0)

    g = NG - 1
    vlast = idx_v.at[g][...]
    for pair in range(2):
        for b in range(2):
            cl = 2 * pair + b
            r0 = 16 * g + 4 * cl
            gather_wait_set(b)
            write_desc(r0, b).start()
        for b in range(2):
            cl = 2 * pair + b
            r0 = 16 * g + 4 * cl
            write_desc(r0, b).wait()
            if pair == 0:
                chunk_gather(vlast, cl + 2, cl + 2)


def _tc_body(idx_ref, row_ref, out_ref):
    out_ref[...] = row_ref[...]


_tc_gather = pl.pallas_call(
    _tc_body,
    grid_spec=pltpu.PrefetchScalarGridSpec(
        num_scalar_prefetch=1,
        grid=(TC_TOKENS,),
        in_specs=[
            pl.BlockSpec((1, 1, HIDDEN), lambda i, idx_ref: (idx_ref[i], 0, 0)),
        ],
        out_specs=pl.BlockSpec((1, 1, HIDDEN), lambda i, idx_ref: (i, 0, 0)),
    ),
    out_shape=jax.ShapeDtypeStruct((TC_TOKENS, 1, HIDDEN), jnp.float32),
)


def kernel(input_ids, word_embeddings):
    b, s = input_ids.shape
    flat = input_ids.reshape(TOKENS)
    idx_sc = flat[TC_TOKENS:].reshape(NW, NG, 16)
    out_sc = _sc_gather(idx_sc, word_embeddings)
    out_tc = _tc_gather(
        flat[:TC_TOKENS], word_embeddings.reshape(VOCAB, 1, HIDDEN)
    ).reshape(TC_TOKENS, HIDDEN)
    out = jnp.concatenate([out_tc, out_sc], axis=0)
    return out.reshape(b, s, HIDDEN)


# speedup vs baseline: 35.8246x; 35.8246x over previous
import functools

import jax
import jax.numpy as jnp
from jax import lax
from jax.experimental import pallas as pl
from jax.experimental.pallas import tpu as pltpu
from jax.experimental.pallas import tpu_sc as plsc

VOCAB = 150528
HIDDEN = 12288
TOKENS = 8192

NC, NS = 2, 16
NW = NC * NS
ROWS_PER_W = TOKENS // NW
NG = ROWS_PER_W // 16

_mesh = plsc.VectorSubcoreMesh(
    core_axis_name="c", subcore_axis_name="s", num_cores=NC, num_subcores=NS
)


@functools.partial(
    pl.kernel,
    mesh=_mesh,
    out_type=jax.ShapeDtypeStruct((TOKENS, HIDDEN), jnp.float32),
    scratch_types=[
        pltpu.VMEM((NG, 16), jnp.int32),
        pltpu.VMEM_SHARED((NS, 2, 4, HIDDEN), jnp.float32),
        [pltpu.SemaphoreType.DMA for _ in range(2)],
        [pltpu.SemaphoreType.DMA for _ in range(2)],
    ],
)
def _sc_gather(idx_hbm, table_hbm, out_hbm, idx_v, spbuf, gsem, wsem):
    cid = lax.axis_index("c")
    sid = lax.axis_index("s")
    wid = sid * NC + cid
    base = wid * ROWS_PER_W
    pltpu.sync_copy(idx_hbm.at[wid], idx_v)

    def gather_start(row, b, j):
        pltpu.make_async_copy(
            table_hbm.at[row], spbuf.at[sid, b, j], gsem[b]
        ).start()

    def gather_wait_set(b):
        pltpu.make_async_copy(
            table_hbm.at[pl.ds(0, 4)], spbuf.at[sid, b], gsem[b]
        ).wait()

    def write_desc(r0, b):
        return pltpu.make_async_copy(
            spbuf.at[sid, b], out_hbm.at[pl.ds(base + r0, 4)], wsem[b]
        )

    def chunk_gather(v, lane_cl, cl):
        b = cl % 2
        for j in range(4):
            gather_start(v[4 * lane_cl + j], b, j)

    v0 = idx_v.at[0][...]
    chunk_gather(v0, 0, 0)
    chunk_gather(v0, 1, 1)

    def body(g, carry):
        v = idx_v.at[g][...]
        vn = idx_v.at[g + 1][...]
        for pair in range(2):
            for b in range(2):
                cl = 2 * pair + b
                r0 = 16 * g + 4 * cl
                gather_wait_set(b)
                write_desc(r0, b).start()
            for b in range(2):
                cl = 2 * pair + b
                r0 = 16 * g + 4 * cl
                write_desc(r0, b).wait()
                if cl < 2:
                    chunk_gather(v, cl + 2, cl + 2)
                else:
                    chunk_gather(vn, cl - 2, cl - 2)
        return carry

    lax.fori_loop(0, NG - 1, body, 0)

    g = NG - 1
    vlast = idx_v.at[g][...]
    for pair in range(2):
        for b in range(2):
            cl = 2 * pair + b
            r0 = 16 * g + 4 * cl
            gather_wait_set(b)
            write_desc(r0, b).start()
        for b in range(2):
            cl = 2 * pair + b
            r0 = 16 * g + 4 * cl
            write_desc(r0, b).wait()
            if pair == 0:
                chunk_gather(vlast, cl + 2, cl + 2)


def kernel(input_ids, word_embeddings):
    b, s = input_ids.shape
    idx = input_ids.reshape(NW, NG, 16)
    out = _sc_gather(idx, word_embeddings)
    return out.reshape(b, s, HIDDEN)
